# baseline (speedup 1.0000x reference)
import functools

import jax
import jax.numpy as jnp
from jax import lax
from jax.experimental import pallas as pl
from jax.experimental.pallas import tpu as pltpu
from jax.experimental.pallas import tpu_sc as plsc


@functools.cache
def _build(B, D, VU, VF):
    info = plsc.get_sparse_core_info()
    NW = info.num_cores * info.num_subcores
    NC = info.num_cores
    b_per_w = B // NW

    mesh = plsc.VectorSubcoreMesh(core_axis_name="c", subcore_axis_name="s")

    @functools.partial(
        pl.kernel,
        mesh=mesh,
        out_type=jax.ShapeDtypeStruct((B, 2 * D), jnp.float32),
        compiler_params=pltpu.CompilerParams(use_tc_tiling_on_sc=False),
        scratch_types=[
            pltpu.VMEM((b_per_w,), jnp.int32),
            pltpu.VMEM((b_per_w,), jnp.int32),
            pltpu.VMEM((b_per_w, D), jnp.float32),
            pltpu.VMEM((b_per_w, D), jnp.float32),
            pltpu.SemaphoreType.DMA,
            pltpu.SemaphoreType.DMA,
        ],
    )
    def k(uid_hbm, eid_hbm, ut_hbm, ft_hbm, out_hbm,
          uidx_v, fidx_v, urows_v, frows_v, sem_u, sem_f):
        wid = lax.axis_index("s") * NC + lax.axis_index("c")
        base = wid * b_per_w
        pltpu.sync_copy(uid_hbm.at[pl.ds(base, b_per_w)], uidx_v)
        pltpu.sync_copy(eid_hbm.at[pl.ds(base, b_per_w)], fidx_v)
        cu = pltpu.async_copy(ut_hbm.at[uidx_v], urows_v, sem_u)
        cf = pltpu.async_copy(ft_hbm.at[fidx_v], frows_v, sem_f)
        cu.wait()
        cf.wait()
        pltpu.sync_copy(urows_v, out_hbm.at[pl.ds(base, b_per_w), pl.ds(0, D)])
        pltpu.sync_copy(frows_v, out_hbm.at[pl.ds(base, b_per_w), pl.ds(D, D)])

    return k


def kernel(userId, emotionId, user_table, feeling_table):
    B = userId.shape[0]
    VU, D = user_table.shape
    VF = feeling_table.shape[0]
    return _build(B, D, VU, VF)(userId, emotionId, user_table, feeling_table)
